# baseline (speedup 1.0000x reference)
import functools

import jax
import jax.numpy as jnp
from jax.experimental import pallas as pl
from jax.experimental.pallas import tpu as pltpu

STATE_DIM = 2048
ACTION_DIM = 512
BATCH = 512

TILE = 512
N_TILES = STATE_DIM // TILE
NSLOTS = 6

_DNT = (((1,), (1,)), ((), ()))

_TILES = [("w1", t) for t in range(N_TILES)] + \
         [("w2", t) for t in range(N_TILES)] + \
         [("wf", t) for t in range(N_TILES)]


def _dot_t(x, w):
    return jax.lax.dot_general(
        x, w, _DNT,
        precision=jax.lax.Precision.DEFAULT,
        preferred_element_type=jnp.float32,
    )


def _body(
    state_hbm, action_hbm, ns_hbm,
    w1_hbm, b1_ref, w2_hbm, b2_ref, wf_hbm, bf_ref,
    out_ref,
    xs_ref, xa_ref, xn_ref, h_ref, pn_ref,
    *scr,
):
    slots = scr[:NSLOTS]
    w1a = scr[NSLOTS:NSLOTS + N_TILES]
    sems = scr[NSLOTS + N_TILES:2 * NSLOTS + N_TILES]
    sems_a = scr[2 * NSLOTS + N_TILES:2 * NSLOTS + 2 * N_TILES]
    sem_s, sem_act, sem_n = scr[2 * NSLOTS + 2 * N_TILES:]

    def tile_copy(idx, slot):
        kind, t = _TILES[idx]
        rows = pl.ds(t * TILE, TILE)
        if kind == "w1":
            src = w1_hbm.at[rows, pl.ds(0, STATE_DIM)]
        elif kind == "w2":
            src = w2_hbm.at[rows, :]
        else:
            src = wf_hbm.at[rows, :]
        return pltpu.make_async_copy(src, slots[slot], sems[slot])

    def w1a_copy(t):
        return pltpu.make_async_copy(
            w1_hbm.at[pl.ds(t * TILE, TILE), pl.ds(STATE_DIM, ACTION_DIM)],
            w1a[t], sems_a[t])

    cp_s = pltpu.make_async_copy(state_hbm, xs_ref, sem_s)
    cp_act = pltpu.make_async_copy(action_hbm, xa_ref, sem_act)
    cp_n = pltpu.make_async_copy(ns_hbm, xn_ref, sem_n)
    cp_s.start()
    cp_act.start()
    w1a_copy(0).start()
    for i in range(NSLOTS):
        tile_copy(i, i).start()
    for t in range(1, N_TILES):
        w1a_copy(t).start()
    cp_n.start()

    sse = jnp.zeros((), jnp.float32)
    for idx in range(len(_TILES)):
        slot = idx % NSLOTS
        tile_copy(idx, slot).wait()
        kind, t = _TILES[idx]
        col = pl.ds(t * TILE, TILE)
        w = slots[slot][...]
        if kind == "w1":
            if t == 0:
                cp_s.wait()
                cp_act.wait()
            w1a_copy(t).wait()
            acc = _dot_t(xs_ref[...], w)
            acc += _dot_t(xa_ref[...], w1a[t][...])
            h_ref[:, col] = jnp.maximum(acc + b1_ref[col][None, :], 0.0)
        elif kind == "w2":
            pn_ref[:, col] = _dot_t(h_ref[...], w) + b2_ref[col][None, :]
        else:
            if t == 0:
                cp_n.wait()
            b = bf_ref[col][None, :]
            fa = jnp.maximum(_dot_t(xn_ref[...], w) + b, 0.0)
            fp = jnp.maximum(_dot_t(pn_ref[...], w) + b, 0.0)
            d = fp - fa
            sse += jnp.sum(d * d)
        nxt = idx + NSLOTS
        if nxt < len(_TILES):
            tile_copy(nxt, slot).start()

    out_ref[...] = sse.reshape(1, 1)


@functools.partial(jax.jit, static_argnames=())
def kernel(state, action, next_state, W1, b1, W2, b2, Wf, bf):
    vmem = functools.partial(pl.BlockSpec, memory_space=pltpu.MemorySpace.VMEM)
    hbm = pl.BlockSpec(memory_space=pltpu.MemorySpace.HBM)
    sse = pl.pallas_call(
        _body,
        in_specs=[
            hbm, hbm, hbm,
            hbm, vmem(),
            hbm, vmem(),
            hbm, vmem(),
        ],
        out_specs=vmem(),
        out_shape=jax.ShapeDtypeStruct((1, 1), jnp.float32),
        scratch_shapes=(
            [pltpu.VMEM((BATCH, STATE_DIM), jnp.float32),
             pltpu.VMEM((BATCH, ACTION_DIM), jnp.float32),
             pltpu.VMEM((BATCH, STATE_DIM), jnp.float32),
             pltpu.VMEM((BATCH, STATE_DIM), jnp.float32),
             pltpu.VMEM((BATCH, STATE_DIM), jnp.float32)]
            + [pltpu.VMEM((TILE, STATE_DIM), jnp.float32)
               for _ in range(NSLOTS)]
            + [pltpu.VMEM((TILE, ACTION_DIM), jnp.float32)
               for _ in range(N_TILES)]
            + [pltpu.SemaphoreType.DMA for _ in range(NSLOTS)]
            + [pltpu.SemaphoreType.DMA for _ in range(N_TILES)]
            + [pltpu.SemaphoreType.DMA for _ in range(3)]
        ),
    )(state, action, next_state, W1, b1, W2, b2, Wf, bf)

    pred_error = sse[0, 0] / jnp.float32(BATCH * STATE_DIM)
    novelty = jnp.float32(1.0)
    curiosity = pred_error * 0.5 + novelty * 0.5
    return jnp.stack([pred_error, novelty, curiosity])


# speedup vs baseline: 1.0128x; 1.0128x over previous
import functools

import jax
import jax.numpy as jnp
from jax.experimental import pallas as pl
from jax.experimental.pallas import tpu as pltpu

STATE_DIM = 2048
ACTION_DIM = 512
BATCH = 512

TILE = 512
HALF = TILE // 2
N_TILES = STATE_DIM // TILE
NSLOTS = 6

_DNT = (((1,), (1,)), ((), ()))

_TILES = [("w1", t) for t in range(N_TILES)] + \
         [("w2", t) for t in range(N_TILES)] + \
         [("wf", t) for t in range(N_TILES)]


def _dot_t(x, w):
    return jax.lax.dot_general(
        x, w, _DNT,
        precision=jax.lax.Precision.DEFAULT,
        preferred_element_type=jnp.float32,
    )


def _body(
    state_ref, action_ref, ns_ref,
    w1_hbm, b1_ref, w2_hbm, b2_ref, wf_hbm, bf_ref,
    out_ref,
    w1a_ref, h_ref, pn_ref, *slot_and_sems,
):
    slots = slot_and_sems[:NSLOTS]
    sems = slot_and_sems[NSLOTS:3 * NSLOTS]
    sem_a = slot_and_sems[3 * NSLOTS]

    def tile_copies(idx, slot):
        kind, t = _TILES[idx]

        def half(i):
            rows = pl.ds(t * TILE + i * HALF, HALF)
            drows = pl.ds(i * HALF, HALF)
            if kind == "w1":
                src = w1_hbm.at[rows, pl.ds(0, STATE_DIM)]
            elif kind == "w2":
                src = w2_hbm.at[rows, :]
            else:
                src = wf_hbm.at[rows, :]
            return pltpu.make_async_copy(
                src, slots[slot].at[drows, :], sems[2 * slot + i])

        return half(0), half(1)

    cp_a = pltpu.make_async_copy(
        w1_hbm.at[:, pl.ds(STATE_DIM, ACTION_DIM)], w1a_ref, sem_a)
    cp_a.start()
    for i in range(NSLOTS):
        a, b = tile_copies(i, i)
        a.start()
        b.start()

    sse = jnp.zeros((), jnp.float32)
    for idx in range(len(_TILES)):
        slot = idx % NSLOTS
        a, b_cp = tile_copies(idx, slot)
        a.wait()
        b_cp.wait()
        kind, t = _TILES[idx]
        col = pl.ds(t * TILE, TILE)
        w = slots[slot][...]
        if kind == "w1":
            if t == 0:
                cp_a.wait()
            acc = _dot_t(state_ref[...], w)
            acc += _dot_t(action_ref[...], w1a_ref[pl.ds(t * TILE, TILE), :])
            h_ref[:, col] = jnp.maximum(acc + b1_ref[col][None, :], 0.0)
        elif kind == "w2":
            pn_ref[:, col] = _dot_t(h_ref[...], w) + b2_ref[col][None, :]
        else:
            b = bf_ref[col][None, :]
            fa = jnp.maximum(_dot_t(ns_ref[...], w) + b, 0.0)
            fp = jnp.maximum(_dot_t(pn_ref[...], w) + b, 0.0)
            d = fp - fa
            sse += jnp.sum(d * d)
        nxt = idx + NSLOTS
        if nxt < len(_TILES):
            a, b_cp = tile_copies(nxt, slot)
            a.start()
            b_cp.start()

    out_ref[...] = sse.reshape(1, 1)


@functools.partial(jax.jit, static_argnames=())
def kernel(state, action, next_state, W1, b1, W2, b2, Wf, bf):
    vmem = functools.partial(pl.BlockSpec, memory_space=pltpu.MemorySpace.VMEM)
    hbm = pl.BlockSpec(memory_space=pltpu.MemorySpace.HBM)
    sse = pl.pallas_call(
        _body,
        in_specs=[
            vmem(), vmem(), vmem(),
            hbm, vmem(),
            hbm, vmem(),
            hbm, vmem(),
        ],
        out_specs=vmem(),
        out_shape=jax.ShapeDtypeStruct((1, 1), jnp.float32),
        scratch_shapes=(
            [pltpu.VMEM((STATE_DIM, ACTION_DIM), jnp.float32)]
            + [pltpu.VMEM((BATCH, STATE_DIM), jnp.float32)] * 2
            + [pltpu.VMEM((TILE, STATE_DIM), jnp.float32)
               for _ in range(NSLOTS)]
            + [pltpu.SemaphoreType.DMA for _ in range(2 * NSLOTS)]
            + [pltpu.SemaphoreType.DMA]
        ),
    )(state, action, next_state, W1, b1, W2, b2, Wf, bf)

    pred_error = sse[0, 0] / jnp.float32(BATCH * STATE_DIM)
    novelty = jnp.float32(1.0)
    curiosity = pred_error * 0.5 + novelty * 0.5
    return jnp.stack([pred_error, novelty, curiosity])
